# baseline (speedup 1.0000x reference)
import functools

import jax
import jax.numpy as jnp
from jax import lax
from jax.experimental import pallas as pl
from jax.experimental.pallas import tpu as pltpu
from jax.experimental.pallas import tpu_sc as plsc

_FIELD_DIM = 100000
_NUM_FIELDS = 26
_EMBED_DIM = 16
_BATCH = 16384

_NC = 2
_NS = 16
_NW = _NC * _NS
_CHUNK = 128
_PER_WORKER = _BATCH // _NW
_NCHUNKS = _PER_WORKER // _CHUNK


def _fm_kernel(x_hbm, emb_hbm, lin_hbm, bias_hbm, out_hbm,
               idx_v, rows_v, lin_v, vbuf_v, out_v, bias_v,
               sem_idx, sem_emb, sem_lin):
    wid = lax.axis_index("s") * _NC + lax.axis_index("c")

    pltpu.sync_copy(bias_hbm, bias_v)
    bias16 = bias_v[...]
    iota16 = lax.iota(jnp.int32, 16)

    for k in range(_NCHUNKS):
        base = wid * _PER_WORKER + k * _CHUNK

        idx_cps = [
            pltpu.async_copy(
                x_hbm.at[f, pl.ds(base, _CHUNK)],
                idx_v.at[pl.ds(f * _CHUNK, _CHUNK)],
                sem_idx,
            )
            for f in range(_NUM_FIELDS)
        ]
        for cp in idx_cps:
            cp.wait()

        def _add_offsets(i, carry):
            f = i // (_CHUNK // 16)
            off = i * 16
            idx_v[pl.ds(off, 16)] = idx_v[pl.ds(off, 16)] + f * _FIELD_DIM
            return carry

        lax.fori_loop(0, _NUM_FIELDS * (_CHUNK // 16), _add_offsets, 0)

        emb_cps = [
            pltpu.async_copy(
                emb_hbm.at[idx_v.at[pl.ds(f * _CHUNK, _CHUNK)]],
                rows_v.at[pl.ds(f * _CHUNK, _CHUNK)],
                sem_emb,
            )
            for f in range(_NUM_FIELDS)
        ]
        lin_cps = [
            pltpu.async_copy(
                lin_hbm.at[idx_v.at[pl.ds(f * _CHUNK, _CHUNK)]],
                lin_v.at[pl.ds(f * _CHUNK, _CHUNK)],
                sem_lin,
            )
            for f in range(_NUM_FIELDS)
        ]
        for cp in emb_cps:
            cp.wait()
        for cp in lin_cps:
            cp.wait()

        def _sample_body(s, carry):
            acc = jnp.zeros((16,), jnp.float32)
            acc2 = jnp.zeros((16,), jnp.float32)
            for f in range(_NUM_FIELDS):
                r = rows_v[f * _CHUNK + s, :]
                acc = acc + r
                acc2 = acc2 + r * r
            v = acc * acc - acc2
            vbuf_v[pl.ds(s * 16, 16)] = v
            return carry

        lax.fori_loop(0, _CHUNK, _sample_body, 0)

        def _group_body(j, carry):
            s0 = j * 16
            lin16 = jnp.zeros((16,), jnp.float32)
            for f in range(_NUM_FIELDS):
                lin16 = lin16 + lin_v[pl.ds(f * _CHUNK + s0, 16)]
            bvec = s0 * 16 + iota16 * 16
            fm16 = jnp.zeros((16,), jnp.float32)
            for d in range(_EMBED_DIM):
                fm16 = fm16 + plsc.load_gather(vbuf_v, [bvec + d])
            out_v[pl.ds(s0, 16)] = 0.5 * fm16 + lin16 + bias16
            return carry

        lax.fori_loop(0, _CHUNK // 16, _group_body, 0)

        pltpu.sync_copy(out_v, out_hbm.at[pl.ds(base, _CHUNK)])


@jax.jit
def _fm_forward(x_t, emb_table, lin_flat, bias):
    kern = functools.partial(
        pl.kernel,
        out_type=jax.ShapeDtypeStruct((_BATCH,), jnp.float32),
        mesh=plsc.VectorSubcoreMesh(core_axis_name="c", subcore_axis_name="s"),
        scratch_types=[
            pltpu.VMEM((_NUM_FIELDS * _CHUNK,), jnp.int32),
            pltpu.VMEM((_NUM_FIELDS * _CHUNK, _EMBED_DIM), jnp.float32),
            pltpu.VMEM((_NUM_FIELDS * _CHUNK,), jnp.float32),
            pltpu.VMEM((_CHUNK * 16,), jnp.float32),
            pltpu.VMEM((_CHUNK,), jnp.float32),
            pltpu.VMEM((16,), jnp.float32),
            pltpu.SemaphoreType.DMA,
            pltpu.SemaphoreType.DMA,
            pltpu.SemaphoreType.DMA,
        ],
        compiler_params=pltpu.CompilerParams(
            needs_layout_passes=False, use_tc_tiling_on_sc=False),
    )(_fm_kernel)
    return kern(x_t, emb_table, lin_flat, bias)


def kernel(x, emb_table, lin_table, bias):
    x_t = x.astype(jnp.int32).T
    lin_flat = lin_table.reshape(-1)
    bias16 = jnp.broadcast_to(bias.astype(jnp.float32), (16,))
    out = _fm_forward(x_t, emb_table, lin_flat, bias16)
    return out
